# baseline (speedup 1.0000x reference)
import functools

import jax
import jax.numpy as jnp
from jax import lax
from jax.experimental import pallas as pl
from jax.experimental.pallas import tpu as pltpu
from jax.experimental.pallas import tpu_sc as plsc

_EDIM = 32
_D = 64
_B = 16384
_NROW = 1000000
_NC = 2
_NS = 16
_NW = _NC * _NS
_BPW = _B // _NW
_UNROLL = 16
_NSEM = 8


_RPS = _B // 2


def _sc_gather_body(uid_hbm, iid_hbm, ut_hbm, it_hbm, ue_hbm, ie_hbm,
                    ids_s, semu, semi):
    cid = lax.axis_index("c")
    base = cid * _RPS

    for ids, table, out, sem in ((uid_hbm, ut_hbm, ue_hbm, semu),
                                 (iid_hbm, it_hbm, ie_hbm, semi)):
        pltpu.sync_copy(ids.at[pl.ds(base, _RPS)], ids_s)

        def fire(g, _, table=table, out=out, sem=sem):
            for u in range(_UNROLL):
                r = g * _UNROLL + u
                rid = ids_s[r]
                pltpu.async_copy(table.at[rid], out.at[base + r], sem)
            return _

        lax.fori_loop(0, _RPS // _UNROLL, fire, None)

    pltpu.make_async_copy(
        ut_hbm.at[pl.ds(0, _RPS)], ue_hbm.at[pl.ds(base, _RPS)], semu).wait()
    pltpu.make_async_copy(
        it_hbm.at[pl.ds(0, _RPS)], ie_hbm.at[pl.ds(base, _RPS)], semi).wait()


_sc_gather = functools.partial(
    pl.kernel,
    out_type=[
        jax.ShapeDtypeStruct((_B, _D), jnp.float32),
        jax.ShapeDtypeStruct((_B, _D), jnp.float32),
    ],
    mesh=plsc.ScalarSubcoreMesh(axis_name="c", num_cores=_NC),
    scratch_types=[
        pltpu.SMEM((_RPS,), jnp.int32),
        pltpu.SemaphoreType.DMA,
        pltpu.SemaphoreType.DMA,
    ],
)(_sc_gather_body)


def _mlp_body(ue_ref, ie_ref, w1u_ref, w1i_ref, b1_ref, w2_ref, b2_ref,
              w3l_ref, w3r_ref, b3_ref, o_ref):
    ue = ue_ref[...]
    ie = ie_ref[...]
    left = ue[:, :_EDIM] * ie[:, :_EDIM]
    h1 = jnp.dot(ue[:, _EDIM:], w1u_ref[...], preferred_element_type=jnp.float32)
    h1 = h1 + jnp.dot(ie[:, _EDIM:], w1i_ref[...], preferred_element_type=jnp.float32)
    h1 = jnp.maximum(h1 + b1_ref[...], 0.0)
    h2 = jnp.dot(h1, w2_ref[...], preferred_element_type=jnp.float32)
    h2 = jnp.maximum(h2 + b2_ref[...], 0.0)
    o = jnp.sum(left * w3l_ref[...], axis=1) + jnp.sum(h2 * w3r_ref[...], axis=1)
    o_ref[...] = o + b3_ref[...]


_BLK = 2048
_GRID = _B // _BLK


def _mlp_head(ue, ie, w1u, w1i, b1, w2, b2, w3l, w3r, b3):
    full = lambda shape: pl.BlockSpec(shape, lambda i: (0, 0))
    return pl.pallas_call(
        _mlp_body,
        grid=(_GRID,),
        in_specs=[
            pl.BlockSpec((_BLK, _D), lambda i: (i, 0)),
            pl.BlockSpec((_BLK, _D), lambda i: (i, 0)),
            full((_EDIM, _EDIM)),
            full((_EDIM, _EDIM)),
            full((1, _EDIM)),
            full((_EDIM, _EDIM // 2)),
            full((1, _EDIM // 2)),
            full((1, _EDIM)),
            full((1, _EDIM // 2)),
            pl.BlockSpec((1,), lambda i: (0,)),
        ],
        out_specs=pl.BlockSpec((_BLK,), lambda i: (i,)),
        out_shape=jax.ShapeDtypeStruct((_B,), jnp.float32),
    )(ue, ie, w1u, w1i, b1, w2, b2, w3l, w3r, b3)


def kernel(user_ids, item_ids, user_table, item_table, W1, b1, W2, b2, W3, b3):
    uid = user_ids.astype(jnp.int32)
    iid = item_ids.astype(jnp.int32)
    ue, ie = _sc_gather(uid, iid, user_table, item_table)
    return _mlp_head(
        ue, ie,
        W1[:, :_EDIM].T, W1[:, _EDIM:].T, b1[None, :],
        W2.T, b2[None, :],
        W3[:, :_EDIM], W3[:, _EDIM:], b3,
    )


# speedup vs baseline: 1.0018x; 1.0018x over previous
import functools

import jax
import jax.numpy as jnp
from jax import lax
from jax.experimental import pallas as pl
from jax.experimental.pallas import tpu as pltpu
from jax.experimental.pallas import tpu_sc as plsc

_EDIM = 32
_D = 64
_B = 16384
_NROW = 1000000
_NC = 2
_NS = 16
_NW = _NC * _NS
_BPW = _B // _NW
_UNROLL = 16
_NSEM = 8


def _sc_gather_body(uid_hbm, iid_hbm, ut_hbm, it_hbm, ue_hbm, ie_hbm,
                    ids_v, semu, semi):
    wid = lax.axis_index("s") * _NC + lax.axis_index("c")
    base = wid * _BPW

    for ids, table, out, sem in ((uid_hbm, ut_hbm, ue_hbm, semu),
                                 (iid_hbm, it_hbm, ie_hbm, semi)):
        pltpu.sync_copy(ids.at[pl.ds(base, _BPW)], ids_v)

        def fire(g, _, table=table, out=out, sem=sem):
            vec = ids_v[pl.ds(pl.multiple_of(g * _UNROLL, _UNROLL), _UNROLL)]
            for u in range(_UNROLL):
                rid = vec[u]
                pltpu.async_copy(
                    table.at[rid], out.at[base + g * _UNROLL + u], sem)
            return _

        lax.fori_loop(0, _BPW // _UNROLL, fire, None)

    pltpu.make_async_copy(
        ut_hbm.at[pl.ds(0, _BPW)], ue_hbm.at[pl.ds(base, _BPW)], semu).wait()
    pltpu.make_async_copy(
        it_hbm.at[pl.ds(0, _BPW)], ie_hbm.at[pl.ds(base, _BPW)], semi).wait()


_sc_gather = functools.partial(
    pl.kernel,
    out_type=[
        jax.ShapeDtypeStruct((_B, _D), jnp.float32),
        jax.ShapeDtypeStruct((_B, _D), jnp.float32),
    ],
    mesh=plsc.VectorSubcoreMesh(core_axis_name="c", subcore_axis_name="s"),
    scratch_types=[
        pltpu.VMEM((_BPW,), jnp.int32),
        pltpu.SemaphoreType.DMA,
        pltpu.SemaphoreType.DMA,
    ],
)(_sc_gather_body)


def _mlp_body(ue_ref, ie_ref, w1u_ref, w1i_ref, b1_ref, w2_ref, b2_ref,
              w3l_ref, w3r_ref, b3_ref, o_ref):
    ue = ue_ref[...]
    ie = ie_ref[...]
    left = ue[:, :_EDIM] * ie[:, :_EDIM]
    h1 = jnp.dot(ue[:, _EDIM:], w1u_ref[...], preferred_element_type=jnp.float32)
    h1 = h1 + jnp.dot(ie[:, _EDIM:], w1i_ref[...], preferred_element_type=jnp.float32)
    h1 = jnp.maximum(h1 + b1_ref[...], 0.0)
    h2 = jnp.dot(h1, w2_ref[...], preferred_element_type=jnp.float32)
    h2 = jnp.maximum(h2 + b2_ref[...], 0.0)
    o = jnp.sum(left * w3l_ref[...], axis=1) + jnp.sum(h2 * w3r_ref[...], axis=1)
    o_ref[...] = o + b3_ref[...]


_BLK = 2048
_GRID = _B // _BLK


def _mlp_head(ue, ie, w1u, w1i, b1, w2, b2, w3l, w3r, b3):
    full = lambda shape: pl.BlockSpec(shape, lambda i: (0, 0))
    return pl.pallas_call(
        _mlp_body,
        grid=(_GRID,),
        in_specs=[
            pl.BlockSpec((_BLK, _D), lambda i: (i, 0)),
            pl.BlockSpec((_BLK, _D), lambda i: (i, 0)),
            full((_EDIM, _EDIM)),
            full((_EDIM, _EDIM)),
            full((1, _EDIM)),
            full((_EDIM, _EDIM // 2)),
            full((1, _EDIM // 2)),
            full((1, _EDIM)),
            full((1, _EDIM // 2)),
            pl.BlockSpec((1,), lambda i: (0,)),
        ],
        out_specs=pl.BlockSpec((_BLK,), lambda i: (i,)),
        out_shape=jax.ShapeDtypeStruct((_B,), jnp.float32),
    )(ue, ie, w1u, w1i, b1, w2, b2, w3l, w3r, b3)


def kernel(user_ids, item_ids, user_table, item_table, W1, b1, W2, b2, W3, b3):
    uid = user_ids.astype(jnp.int32)
    iid = item_ids.astype(jnp.int32)
    ue, ie = _sc_gather(uid, iid, user_table, item_table)
    return _mlp_head(
        ue, ie,
        W1[:, :_EDIM].T, W1[:, _EDIM:].T, b1[None, :],
        W2.T, b2[None, :],
        W3[:, :_EDIM], W3[:, _EDIM:], b3,
    )


# speedup vs baseline: 1.1300x; 1.1279x over previous
import functools

import jax
import jax.numpy as jnp
from jax import lax
from jax.experimental import pallas as pl
from jax.experimental.pallas import tpu as pltpu
from jax.experimental.pallas import tpu_sc as plsc

_EDIM = 32
_D = 64
_B = 16384
_NC = 2
_NS = 16
_NW = _NC * _NS
_UNROLL = 16

_BT = 8192
_BSC = _B - _BT
_RPW = _BSC // _NW
_SSTREAM = 160
_SDMA = _RPW - _SSTREAM


def _tc_gather_body(ids_ref, ut_any, it_any, oue_any, oie_any, semu, semi):
    def fire(g, _):
        for u in range(8):
            r = g * 8 + u
            pltpu.make_async_copy(
                ut_any.at[ids_ref[r]], oue_any.at[r], semu).start()
            pltpu.make_async_copy(
                it_any.at[ids_ref[_BT + r]], oie_any.at[r], semi).start()
        return _

    lax.fori_loop(0, _BT // 8, fire, None)
    pltpu.make_async_copy(
        ut_any.at[pl.ds(0, _BT)], oue_any.at[pl.ds(0, _BT)], semu).wait()
    pltpu.make_async_copy(
        it_any.at[pl.ds(0, _BT)], oie_any.at[pl.ds(0, _BT)], semi).wait()


def _tc_gather(ids2, user_table, item_table):
    grid_spec = pltpu.PrefetchScalarGridSpec(
        num_scalar_prefetch=1,
        grid=(1,),
        in_specs=[
            pl.BlockSpec(memory_space=pl.ANY),
            pl.BlockSpec(memory_space=pl.ANY),
        ],
        out_specs=[
            pl.BlockSpec(memory_space=pl.ANY),
            pl.BlockSpec(memory_space=pl.ANY),
        ],
        scratch_shapes=[pltpu.SemaphoreType.DMA, pltpu.SemaphoreType.DMA],
    )
    return pl.pallas_call(
        _tc_gather_body,
        grid_spec=grid_spec,
        out_shape=[
            jax.ShapeDtypeStruct((_BT, _D), jnp.float32),
            jax.ShapeDtypeStruct((_BT, _D), jnp.float32),
        ],
    )(ids2, user_table, item_table)


def _sc_gather_body(uid_hbm, iid_hbm, ut_hbm, it_hbm, ue_hbm, ie_hbm,
                    ids_v, selu_v, seli_v, semu, semi, semud, semid):
    wid = lax.axis_index("s") * _NC + lax.axis_index("c")
    base = wid * _RPW

    for ids, table, out, sel, sem, semd in (
        (uid_hbm, ut_hbm, ue_hbm, selu_v, semu, semud),
        (iid_hbm, it_hbm, ie_hbm, seli_v, semi, semid),
    ):
        pltpu.sync_copy(ids.at[pl.ds(base, _RPW)], ids_v)

        def fire_stream(g, _, table=table, sel=sel, sem=sem):
            vec = ids_v[pl.ds(pl.multiple_of(g * _UNROLL, _UNROLL), _UNROLL)]
            for u in range(_UNROLL):
                pltpu.async_copy(
                    table.at[vec[u]], sel.at[g * _UNROLL + u], sem)
            return _

        def fire_dma(g, _, table=table, out=out, semd=semd):
            vec = ids_v[pl.ds(pl.multiple_of(g * _UNROLL, _UNROLL), _UNROLL)]
            for u in range(_UNROLL):
                pltpu.async_copy(
                    table.at[vec[u]], out.at[base + g * _UNROLL + u], semd)
            return _

        lax.fori_loop(0, _SSTREAM // _UNROLL, fire_stream, None)
        lax.fori_loop(_SSTREAM // _UNROLL, _RPW // _UNROLL, fire_dma, None)

    for out, sel, sem, semd in ((ue_hbm, selu_v, semu, semud),
                                (ie_hbm, seli_v, semi, semid)):
        pltpu.make_async_copy(
            out.at[pl.ds(0, _SSTREAM)], sel, sem).wait()
        pltpu.sync_copy(sel, out.at[pl.ds(base, _SSTREAM)])
        pltpu.make_async_copy(
            out.at[pl.ds(0, _SDMA)],
            out.at[pl.ds(_SSTREAM, _SDMA)], semd).wait()


_sc_gather = functools.partial(
    pl.kernel,
    out_type=[
        jax.ShapeDtypeStruct((_BSC, _D), jnp.float32),
        jax.ShapeDtypeStruct((_BSC, _D), jnp.float32),
    ],
    mesh=plsc.VectorSubcoreMesh(core_axis_name="c", subcore_axis_name="s"),
    scratch_types=[
        pltpu.VMEM((_RPW,), jnp.int32),
        pltpu.VMEM((_SSTREAM, _D), jnp.float32),
        pltpu.VMEM((_SSTREAM, _D), jnp.float32),
        pltpu.SemaphoreType.DMA,
        pltpu.SemaphoreType.DMA,
        pltpu.SemaphoreType.DMA,
        pltpu.SemaphoreType.DMA,
    ],
)(_sc_gather_body)


def _mlp_body(ue_ref, ie_ref, w1u_ref, w1i_ref, b1_ref, w2_ref, b2_ref,
              w3l_ref, w3r_ref, b3_ref, o_ref):
    ue = ue_ref[...]
    ie = ie_ref[...]
    left = ue[:, :_EDIM] * ie[:, :_EDIM]
    h1 = jnp.dot(ue[:, _EDIM:], w1u_ref[...], preferred_element_type=jnp.float32)
    h1 = h1 + jnp.dot(ie[:, _EDIM:], w1i_ref[...], preferred_element_type=jnp.float32)
    h1 = jnp.maximum(h1 + b1_ref[...], 0.0)
    h2 = jnp.dot(h1, w2_ref[...], preferred_element_type=jnp.float32)
    h2 = jnp.maximum(h2 + b2_ref[...], 0.0)
    o = jnp.sum(left * w3l_ref[...], axis=1) + jnp.sum(h2 * w3r_ref[...], axis=1)
    o_ref[...] = o + b3_ref[...]


_BLK = 2048
_GRID = _B // _BLK


def _mlp_head(ue, ie, w1u, w1i, b1, w2, b2, w3l, w3r, b3):
    full = lambda shape: pl.BlockSpec(shape, lambda i: (0, 0))
    return pl.pallas_call(
        _mlp_body,
        grid=(_GRID,),
        in_specs=[
            pl.BlockSpec((_BLK, _D), lambda i: (i, 0)),
            pl.BlockSpec((_BLK, _D), lambda i: (i, 0)),
            full((_EDIM, _EDIM)),
            full((_EDIM, _EDIM)),
            full((1, _EDIM)),
            full((_EDIM, _EDIM // 2)),
            full((1, _EDIM // 2)),
            full((1, _EDIM)),
            full((1, _EDIM // 2)),
            pl.BlockSpec((1,), lambda i: (0,)),
        ],
        out_specs=pl.BlockSpec((_BLK,), lambda i: (i,)),
        out_shape=jax.ShapeDtypeStruct((_B,), jnp.float32),
    )(ue, ie, w1u, w1i, b1, w2, b2, w3l, w3r, b3)


def kernel(user_ids, item_ids, user_table, item_table, W1, b1, W2, b2, W3, b3):
    uid = user_ids.astype(jnp.int32)
    iid = item_ids.astype(jnp.int32)
    ue_sc, ie_sc = _sc_gather(uid[_BT:], iid[_BT:], user_table, item_table)
    ids2 = jnp.concatenate([uid[:_BT], iid[:_BT]])
    ue_tc, ie_tc = _tc_gather(ids2, user_table, item_table)
    ue = jnp.concatenate([ue_tc, ue_sc])
    ie = jnp.concatenate([ie_tc, ie_sc])
    return _mlp_head(
        ue, ie,
        W1[:, :_EDIM].T, W1[:, _EDIM:].T, b1[None, :],
        W2.T, b2[None, :],
        W3[:, :_EDIM], W3[:, _EDIM:], b3,
    )


# speedup vs baseline: 1.3497x; 1.1944x over previous
import functools

import jax
import jax.numpy as jnp
from jax import lax
from jax.experimental import pallas as pl
from jax.experimental.pallas import tpu as pltpu
from jax.experimental.pallas import tpu_sc as plsc

_EDIM = 32
_D = 64
_B = 16384
_NC = 2
_NS = 16
_NW = _NC * _NS
_UNROLL = 16

_BT = 0
_BSC = _B - _BT
_RPW = _BSC // _NW
_SSTREAM = 320
_SDMA = _RPW - _SSTREAM


def _tc_gather_body(ids_ref, ut_any, it_any, oue_any, oie_any, semu, semi):
    def fire(g, _):
        for u in range(8):
            r = g * 8 + u
            pltpu.make_async_copy(
                ut_any.at[ids_ref[r]], oue_any.at[r], semu).start()
            pltpu.make_async_copy(
                it_any.at[ids_ref[_BT + r]], oie_any.at[r], semi).start()
        return _

    lax.fori_loop(0, _BT // 8, fire, None)
    pltpu.make_async_copy(
        ut_any.at[pl.ds(0, _BT)], oue_any.at[pl.ds(0, _BT)], semu).wait()
    pltpu.make_async_copy(
        it_any.at[pl.ds(0, _BT)], oie_any.at[pl.ds(0, _BT)], semi).wait()


def _tc_gather(ids2, user_table, item_table):
    grid_spec = pltpu.PrefetchScalarGridSpec(
        num_scalar_prefetch=1,
        grid=(1,),
        in_specs=[
            pl.BlockSpec(memory_space=pl.ANY),
            pl.BlockSpec(memory_space=pl.ANY),
        ],
        out_specs=[
            pl.BlockSpec(memory_space=pl.ANY),
            pl.BlockSpec(memory_space=pl.ANY),
        ],
        scratch_shapes=[pltpu.SemaphoreType.DMA, pltpu.SemaphoreType.DMA],
    )
    return pl.pallas_call(
        _tc_gather_body,
        grid_spec=grid_spec,
        out_shape=[
            jax.ShapeDtypeStruct((_BT, _D), jnp.float32),
            jax.ShapeDtypeStruct((_BT, _D), jnp.float32),
        ],
    )(ids2, user_table, item_table)


def _sc_gather_body(uid_hbm, iid_hbm, ut_hbm, it_hbm, ue_hbm, ie_hbm,
                    ids_v, selu_v, seli_v, semu, semi, semud, semid):
    wid = lax.axis_index("s") * _NC + lax.axis_index("c")
    base = wid * _RPW

    for ids, table, out, sel, sem, semd in (
        (uid_hbm, ut_hbm, ue_hbm, selu_v, semu, semud),
        (iid_hbm, it_hbm, ie_hbm, seli_v, semi, semid),
    ):
        pltpu.sync_copy(ids.at[pl.ds(base, _RPW)], ids_v)

        def fire_stream(g, _, table=table, sel=sel, sem=sem):
            vec = ids_v[pl.ds(pl.multiple_of(g * _UNROLL, _UNROLL), _UNROLL)]
            for u in range(_UNROLL):
                pltpu.async_copy(
                    table.at[vec[u]], sel.at[g * _UNROLL + u], sem)
            return _

        def fire_dma(g, _, table=table, out=out, semd=semd):
            vec = ids_v[pl.ds(pl.multiple_of(g * _UNROLL, _UNROLL), _UNROLL)]
            for u in range(_UNROLL):
                pltpu.async_copy(
                    table.at[vec[u]], out.at[base + g * _UNROLL + u], semd)
            return _

        lax.fori_loop(0, _SSTREAM // _UNROLL, fire_stream, None)
        lax.fori_loop(_SSTREAM // _UNROLL, _RPW // _UNROLL, fire_dma, None)

    for out, sel, sem, semd in ((ue_hbm, selu_v, semu, semud),
                                (ie_hbm, seli_v, semi, semid)):
        pltpu.make_async_copy(
            out.at[pl.ds(0, _SSTREAM)], sel, sem).wait()
        pltpu.sync_copy(sel, out.at[pl.ds(base, _SSTREAM)])
        pltpu.make_async_copy(
            out.at[pl.ds(0, _SDMA)],
            out.at[pl.ds(_SSTREAM, _SDMA)], semd).wait()


_sc_gather = functools.partial(
    pl.kernel,
    out_type=[
        jax.ShapeDtypeStruct((_BSC, _D), jnp.float32),
        jax.ShapeDtypeStruct((_BSC, _D), jnp.float32),
    ],
    mesh=plsc.VectorSubcoreMesh(core_axis_name="c", subcore_axis_name="s"),
    scratch_types=[
        pltpu.VMEM((_RPW,), jnp.int32),
        pltpu.VMEM((_SSTREAM, _D), jnp.float32),
        pltpu.VMEM((_SSTREAM, _D), jnp.float32),
        pltpu.SemaphoreType.DMA,
        pltpu.SemaphoreType.DMA,
        pltpu.SemaphoreType.DMA,
        pltpu.SemaphoreType.DMA,
    ],
)(_sc_gather_body)


def _mlp_body(ue_ref, ie_ref, w1u_ref, w1i_ref, b1_ref, w2_ref, b2_ref,
              w3l_ref, w3r_ref, b3_ref, o_ref):
    ue = ue_ref[...]
    ie = ie_ref[...]
    left = ue[:, :_EDIM] * ie[:, :_EDIM]
    h1 = jnp.dot(ue[:, _EDIM:], w1u_ref[...], preferred_element_type=jnp.float32)
    h1 = h1 + jnp.dot(ie[:, _EDIM:], w1i_ref[...], preferred_element_type=jnp.float32)
    h1 = jnp.maximum(h1 + b1_ref[...], 0.0)
    h2 = jnp.dot(h1, w2_ref[...], preferred_element_type=jnp.float32)
    h2 = jnp.maximum(h2 + b2_ref[...], 0.0)
    o = jnp.sum(left * w3l_ref[...], axis=1) + jnp.sum(h2 * w3r_ref[...], axis=1)
    o_ref[...] = o + b3_ref[...]


_BLK = 2048
_GRID = _B // _BLK


def _mlp_head(ue, ie, w1u, w1i, b1, w2, b2, w3l, w3r, b3):
    full = lambda shape: pl.BlockSpec(shape, lambda i: (0, 0))
    return pl.pallas_call(
        _mlp_body,
        grid=(_GRID,),
        in_specs=[
            pl.BlockSpec((_BLK, _D), lambda i: (i, 0)),
            pl.BlockSpec((_BLK, _D), lambda i: (i, 0)),
            full((_EDIM, _EDIM)),
            full((_EDIM, _EDIM)),
            full((1, _EDIM)),
            full((_EDIM, _EDIM // 2)),
            full((1, _EDIM // 2)),
            full((1, _EDIM)),
            full((1, _EDIM // 2)),
            pl.BlockSpec((1,), lambda i: (0,)),
        ],
        out_specs=pl.BlockSpec((_BLK,), lambda i: (i,)),
        out_shape=jax.ShapeDtypeStruct((_B,), jnp.float32),
    )(ue, ie, w1u, w1i, b1, w2, b2, w3l, w3r, b3)


def kernel(user_ids, item_ids, user_table, item_table, W1, b1, W2, b2, W3, b3):
    uid = user_ids.astype(jnp.int32)
    iid = item_ids.astype(jnp.int32)
    ue_sc, ie_sc = _sc_gather(uid[_BT:], iid[_BT:], user_table, item_table)
    if _BT:
        ids2 = jnp.concatenate([uid[:_BT], iid[:_BT]])
        ue_tc, ie_tc = _tc_gather(ids2, user_table, item_table)
        ue = jnp.concatenate([ue_tc, ue_sc])
        ie = jnp.concatenate([ie_tc, ie_sc])
    else:
        ue, ie = ue_sc, ie_sc
    return _mlp_head(
        ue, ie,
        W1[:, :_EDIM].T, W1[:, _EDIM:].T, b1[None, :],
        W2.T, b2[None, :],
        W3[:, :_EDIM], W3[:, _EDIM:], b3,
    )
